# baseline (speedup 1.0000x reference)
import functools

import jax
import jax.numpy as jnp
import numpy as np
from jax import lax
from jax.experimental import pallas as pl
from jax.experimental.pallas import tpu as pltpu
from jax.experimental.pallas import tpu_sc as plsc

BATCH = 4096
N_SAMPLE = 100
EMB_DIM = 128
NUM_ITEMS = 100000
TOTAL = BATCH * N_SAMPLE

NW = 32
ROWS_W = BATCH // NW
NBUF = 2


def _derive_lower_key():
    def rotl(x, d):
        return np.uint32((x << np.uint32(d)) | (x >> np.uint32(32 - d)))

    def tf2x32(k0, k1, x0, x1):
        ks2 = np.uint32(k0 ^ k1 ^ np.uint32(0x1BD11BDA))
        x0 = np.uint32(x0 + k0)
        x1 = np.uint32(x1 + k1)
        rots = ([13, 15, 26, 6], [17, 29, 16, 24])
        ks = [k1, ks2, k0, k1, ks2, k0]
        for i in range(5):
            for r in rots[i % 2]:
                x0 = np.uint32(x0 + x1)
                x1 = rotl(x1, r)
                x1 = np.uint32(x0 ^ x1)
            x0 = np.uint32(x0 + ks[i])
            x1 = np.uint32(ks[i + 1] + np.uint32(i + 1) + x1)
        return x0, x1

    b0 = tf2x32(np.uint32(0), np.uint32(42), np.uint32(0), np.uint32(1))
    return int(b0[0]), int(b0[1])


_K0, _K1 = _derive_lower_key()

_ROT_A = (13, 15, 26, 6)
_ROT_B = (17, 29, 16, 24)


def _u32(x):
    return jnp.uint32(x)


def _threefry_bits(x1):
    ks0 = _u32(_K0)
    ks1 = _u32(_K1)
    ks2 = _u32(_K0 ^ _K1 ^ 0x1BD11BDA)
    v0 = jnp.full(x1.shape, ks0, jnp.uint32)
    v1 = x1 + ks1

    def group(v0, v1, rots):
        for r in rots:
            v0 = v0 + v1
            v1 = (v1 << _u32(r)) | (v1 >> _u32(32 - r))
            v1 = v0 ^ v1
        return v0, v1

    v0, v1 = group(v0, v1, _ROT_A)
    v0 = v0 + ks1
    v1 = v1 + ks2 + _u32(1)
    v0, v1 = group(v0, v1, _ROT_B)
    v0 = v0 + ks2
    v1 = v1 + ks0 + _u32(2)
    v0, v1 = group(v0, v1, _ROT_A)
    v0 = v0 + ks0
    v1 = v1 + ks1 + _u32(3)
    v0, v1 = group(v0, v1, _ROT_B)
    v0 = v0 + ks1
    v1 = v1 + ks2 + _u32(4)
    v0, v1 = group(v0, v1, _ROT_A)
    v0 = v0 + ks2
    v1 = v1 + ks0 + _u32(5)
    return v0 ^ v1


def _mod_span(x):
    for k in range(15, -1, -1):
        t = _u32(NUM_ITEMS << k)
        x = jnp.where(x >= t, x - t, x)
    return x


def _ids_vecs(b0, j, pos_ref):
    lane100 = lax.iota(jnp.uint32, 16) * _u32(N_SAMPLE)
    out = []
    for v in range(ROWS_W // 16):
        base = ((b0 + v * 16) * N_SAMPLE + j).astype(jnp.uint32)
        bits = _threefry_bits(base + lane100)
        neg = (_mod_span(bits) + _u32(1)).astype(jnp.int32)
        pos = pos_ref[pl.ds(v * 16, 16)]
        fixed = jnp.where(neg >= NUM_ITEMS - 1, 1, neg + 1)
        out.append(jnp.where(neg == pos, fixed, neg))
    return out


def _sc_body(pos_hbm, emb_hbm, ids_hbm, out_hbm,
             pos_v, idx_v, rows_v, gsem, osem, isem):
    wid = lax.axis_index("s") * 2 + lax.axis_index("c")
    row0 = pl.multiple_of(wid * ROWS_W, ROWS_W)

    pltpu.sync_copy(pos_hbm.at[pl.ds(row0, ROWS_W)], pos_v)

    def bsl():
        return pl.ds(row0, ROWS_W)

    def chunk(j, slot):
        del slot
        ids = _ids_vecs(row0, j, pos_v)
        for v in range(ROWS_W // 16):
            idx_v[j, pl.ds(v * 16, 16)] = ids[v]

    def fire(j, slot):
        pltpu.async_copy(emb_hbm.at[idx_v.at[j]], rows_v.at[slot],
                         gsem.at[slot])

    def wait_gather(j, slot):
        pltpu.make_async_copy(emb_hbm.at[idx_v.at[j]], rows_v.at[slot],
                              gsem.at[slot]).wait()

    def out_copy(j, slot):
        pltpu.async_copy(rows_v.at[slot], out_hbm.at[j, bsl()], osem.at[slot])

    def wait_out(j, slot):
        pltpu.make_async_copy(rows_v.at[slot], out_hbm.at[j, bsl()],
                              osem.at[slot]).wait()

    for b in range(NBUF):
        j = jnp.int32(b)
        chunk(j, b)
        fire(j, b)

    def body(g, _):
        for b in range(NBUF):
            j = g * NBUF + b
            wait_gather(j - NBUF, b)
            out_copy(j - NBUF, b)
            chunk(j, b)
            wait_out(j - NBUF, b)
            fire(j, b)
        return _

    lax.fori_loop(1, N_SAMPLE // NBUF, body, 0, unroll=False)

    pltpu.async_copy(idx_v, ids_hbm.at[:, bsl()], isem)

    for b in range(NBUF):
        j = jnp.int32(N_SAMPLE - NBUF + b)
        wait_gather(j, b)
        out_copy(j, b)
        wait_out(j, b)

    pltpu.make_async_copy(idx_v, ids_hbm.at[:, bsl()], isem).wait()


@functools.partial(jax.jit, static_argnums=())
def _run(positive_ids, item_emb):
    mesh = plsc.VectorSubcoreMesh(core_axis_name="c", subcore_axis_name="s")
    kfn = pl.kernel(
        _sc_body,
        out_type=[
            jax.ShapeDtypeStruct((N_SAMPLE, BATCH), jnp.int32),
            jax.ShapeDtypeStruct((N_SAMPLE, BATCH, EMB_DIM), jnp.float32),
        ],
        mesh=mesh,
        compiler_params=pltpu.CompilerParams(needs_layout_passes=False,
                                             use_tc_tiling_on_sc=True),
        scratch_types=[
            pltpu.VMEM((ROWS_W,), jnp.int32),
            pltpu.VMEM((N_SAMPLE, ROWS_W), jnp.int32),
            pltpu.VMEM((NBUF, ROWS_W, EMB_DIM), jnp.float32),
            pltpu.SemaphoreType.DMA((NBUF,)),
            pltpu.SemaphoreType.DMA((NBUF,)),
            pltpu.SemaphoreType.DMA,
        ],
    )
    return kfn(positive_ids, item_emb)


def kernel(positive_ids, num_to_sample, item_emb):
    del num_to_sample
    ids_t, emb_t = _run(positive_ids, item_emb)
    return (jnp.transpose(ids_t, (1, 0)), jnp.transpose(emb_t, (1, 0, 2)))


# speedup vs baseline: 1.0070x; 1.0070x over previous
import functools

import jax
import jax.numpy as jnp
import numpy as np
from jax import lax
from jax.experimental import pallas as pl
from jax.experimental.pallas import tpu as pltpu
from jax.experimental.pallas import tpu_sc as plsc

BATCH = 4096
N_SAMPLE = 100
EMB_DIM = 128
NUM_ITEMS = 100000
TOTAL = BATCH * N_SAMPLE

NW = 32
ROWS_W = BATCH // NW
NBUF = 2


def _derive_lower_key():
    def rotl(x, d):
        return np.uint32((x << np.uint32(d)) | (x >> np.uint32(32 - d)))

    def tf2x32(k0, k1, x0, x1):
        ks2 = np.uint32(k0 ^ k1 ^ np.uint32(0x1BD11BDA))
        x0 = np.uint32(x0 + k0)
        x1 = np.uint32(x1 + k1)
        rots = ([13, 15, 26, 6], [17, 29, 16, 24])
        ks = [k1, ks2, k0, k1, ks2, k0]
        for i in range(5):
            for r in rots[i % 2]:
                x0 = np.uint32(x0 + x1)
                x1 = rotl(x1, r)
                x1 = np.uint32(x0 ^ x1)
            x0 = np.uint32(x0 + ks[i])
            x1 = np.uint32(ks[i + 1] + np.uint32(i + 1) + x1)
        return x0, x1

    b0 = tf2x32(np.uint32(0), np.uint32(42), np.uint32(0), np.uint32(1))
    return int(b0[0]), int(b0[1])


_K0, _K1 = _derive_lower_key()

_ROT_A = (13, 15, 26, 6)
_ROT_B = (17, 29, 16, 24)


def _u32(x):
    return jnp.uint32(x)


def _threefry_bits(x1):
    ks0 = _u32(_K0)
    ks1 = _u32(_K1)
    ks2 = _u32(_K0 ^ _K1 ^ 0x1BD11BDA)
    v0 = jnp.full(x1.shape, ks0, jnp.uint32)
    v1 = x1 + ks1

    def group(v0, v1, rots):
        for r in rots:
            v0 = v0 + v1
            v1 = (v1 << _u32(r)) | (v1 >> _u32(32 - r))
            v1 = v0 ^ v1
        return v0, v1

    v0, v1 = group(v0, v1, _ROT_A)
    v0 = v0 + ks1
    v1 = v1 + ks2 + _u32(1)
    v0, v1 = group(v0, v1, _ROT_B)
    v0 = v0 + ks2
    v1 = v1 + ks0 + _u32(2)
    v0, v1 = group(v0, v1, _ROT_A)
    v0 = v0 + ks0
    v1 = v1 + ks1 + _u32(3)
    v0, v1 = group(v0, v1, _ROT_B)
    v0 = v0 + ks1
    v1 = v1 + ks2 + _u32(4)
    v0, v1 = group(v0, v1, _ROT_A)
    v0 = v0 + ks2
    v1 = v1 + ks0 + _u32(5)
    return v0 ^ v1


def _mod_span(x):
    for k in range(15, -1, -1):
        t = _u32(NUM_ITEMS << k)
        x = jnp.where(x >= t, x - t, x)
    return x


def _ids_vecs(b0, j, pos_ref):
    lane100 = lax.iota(jnp.uint32, 16) * _u32(N_SAMPLE)
    out = []
    for v in range(ROWS_W // 16):
        base = ((b0 + v * 16) * N_SAMPLE + j).astype(jnp.uint32)
        bits = _threefry_bits(base + lane100)
        neg = (_mod_span(bits) + _u32(1)).astype(jnp.int32)
        pos = pos_ref[pl.ds(v * 16, 16)]
        fixed = jnp.where(neg >= NUM_ITEMS - 1, 1, neg + 1)
        out.append(jnp.where(neg == pos, fixed, neg))
    return out


def _sc_body(pos_hbm, emb_hbm, ids_hbm, out_hbm,
             pos_v, idx_v, rows_v, gsem, osem, isem):
    wid = lax.axis_index("s") * 2 + lax.axis_index("c")
    row0 = pl.multiple_of(wid * ROWS_W, ROWS_W)

    pltpu.sync_copy(pos_hbm.at[pl.ds(row0, ROWS_W)], pos_v)

    def bsl():
        return pl.ds(row0, ROWS_W)

    def chunk(j, slot):
        ids = _ids_vecs(row0, j, pos_v)
        for v in range(ROWS_W // 16):
            idx_v[slot, pl.ds(v * 16, 16)] = ids[v]

    def fire(j, slot):
        pltpu.async_copy(emb_hbm.at[idx_v.at[slot]], rows_v.at[slot],
                         gsem.at[slot])
        pltpu.async_copy(idx_v.at[slot], ids_hbm.at[j, bsl()], isem.at[slot])

    def wait_gather(slot):
        pltpu.make_async_copy(emb_hbm.at[idx_v.at[slot]], rows_v.at[slot],
                              gsem.at[slot]).wait()

    def out_copy(j, slot):
        pltpu.async_copy(rows_v.at[slot], out_hbm.at[j, bsl()], osem.at[slot])

    def wait_out(j, slot):
        pltpu.make_async_copy(rows_v.at[slot], out_hbm.at[j, bsl()],
                              osem.at[slot]).wait()

    def wait_ids(j, slot):
        pltpu.make_async_copy(idx_v.at[slot], ids_hbm.at[j, bsl()],
                              isem.at[slot]).wait()

    for b in range(NBUF):
        j = jnp.int32(b)
        chunk(j, b)
        fire(j, b)

    def body(g, _):
        for b in range(NBUF):
            j = g * NBUF + b
            wait_gather(b)
            out_copy(j - NBUF, b)
            wait_ids(j - NBUF, b)
            chunk(j, b)
            wait_out(j - NBUF, b)
            fire(j, b)
        return _

    lax.fori_loop(1, N_SAMPLE // NBUF, body, 0, unroll=False)

    for b in range(NBUF):
        j = jnp.int32(N_SAMPLE - NBUF + b)
        wait_gather(b)
        out_copy(j, b)
        wait_ids(j, b)
        wait_out(j, b)


@functools.partial(jax.jit, static_argnums=())
def _run(positive_ids, item_emb):
    mesh = plsc.VectorSubcoreMesh(core_axis_name="c", subcore_axis_name="s")
    kfn = pl.kernel(
        _sc_body,
        out_type=[
            jax.ShapeDtypeStruct((N_SAMPLE, BATCH), jnp.int32),
            jax.ShapeDtypeStruct((N_SAMPLE, BATCH, EMB_DIM), jnp.float32),
        ],
        mesh=mesh,
        compiler_params=pltpu.CompilerParams(needs_layout_passes=False,
                                             use_tc_tiling_on_sc=True),
        scratch_types=[
            pltpu.VMEM((ROWS_W,), jnp.int32),
            pltpu.VMEM((NBUF, ROWS_W), jnp.int32),
            pltpu.VMEM((NBUF, ROWS_W, EMB_DIM), jnp.float32),
            pltpu.SemaphoreType.DMA((NBUF,)),
            pltpu.SemaphoreType.DMA((NBUF,)),
            pltpu.SemaphoreType.DMA((NBUF,)),
        ],
    )
    return kfn(positive_ids, item_emb)


def kernel(positive_ids, num_to_sample, item_emb):
    del num_to_sample
    ids_t, emb_t = _run(positive_ids, item_emb)
    return (jnp.transpose(ids_t, (1, 0)), jnp.transpose(emb_t, (1, 0, 2)))
